# baseline (speedup 1.0000x reference)
import jax
import jax.numpy as jnp
from jax import lax
from jax.experimental import pallas as pl
from jax.experimental.pallas import tpu as pltpu
from jax.experimental.pallas import tpu_sc as plsc

D_MODEL = 64
SCALE = 8.0
NC, NS = 2, 16
NW = NC * NS
CHUNK = 128
LANES = 16
NBUF = 4
AHEAD = 2


def _emb_body(x_hbm, lut_hbm, out_hbm, *bufs):
    raw = bufs[0:NBUF]
    kid = bufs[NBUF:2 * NBUF]
    par = bufs[2 * NBUF:3 * NBUF]
    gbuf = bufs[3 * NBUF:4 * NBUF]
    sbuf = bufs[4 * NBUF:5 * NBUF]
    isem = bufs[5 * NBUF:6 * NBUF]
    gsem = bufs[6 * NBUF:7 * NBUF]
    wsem = bufs[7 * NBUF:8 * NBUF]

    wid = lax.axis_index("s") * NC + lax.axis_index("c")
    n_chunks = x_hbm.shape[0]
    col0 = wid * CHUNK

    def idx_src(j):
        return x_hbm.at[j, pl.ds(col0, CHUNK)]

    def out_dst(j):
        return out_hbm.at[pl.ds(j * D_MODEL, D_MODEL), pl.ds(col0, CHUNK)]

    def prep(b):
        for g in range(CHUNK // LANES):
            s = pl.ds(g * LANES, LANES)
            v = raw[b][s]
            kid[b][s] = lax.shift_right_logical(v, 1)
            par[b][s] = lax.shift_left(lax.bitwise_and(v, 1), 6)

    def slab(b):
        @pl.loop(0, CHUNK // LANES)
        def _g(g):
            s = pl.ds(g * LANES, LANES)
            rows = jax.lax.iota(jnp.int32, LANES) + g * LANES
            pbase = par[b][s]

            @pl.loop(0, D_MODEL, unroll=8)
            def _d(d):
                vals = plsc.load_gather(gbuf[b], [rows, pbase + d])
                sbuf[b][d, s] = vals * SCALE

    def unit(j, b, head, tail):
        jn = j + AHEAD
        bn = (b + AHEAD) % NBUF
        bf = (b + AHEAD + 1) % NBUF
        if not tail:
            pltpu.make_async_copy(idx_src(jn), raw[bn], isem[bn]).wait()
            prep(bn)
            if not head:
                pltpu.make_async_copy(sbuf[bn], out_dst(jn - NBUF), wsem[bn]).wait()
            pltpu.async_copy(lut_hbm.at[kid[bn]], gbuf[bn], gsem[bn])
            jf = jn + 1
            if not (isinstance(jf, int) and jf >= n_chunks):
                pltpu.async_copy(idx_src(jf), raw[bf], isem[bf])
        pltpu.make_async_copy(lut_hbm.at[kid[b]], gbuf[b], gsem[b]).wait()
        slab(b)
        pltpu.async_copy(sbuf[b], out_dst(j), wsem[b])

    pltpu.async_copy(idx_src(0), raw[0], isem[0])
    for k in range(AHEAD):
        pltpu.make_async_copy(idx_src(k), raw[k], isem[k]).wait()
        prep(k)
        pltpu.async_copy(lut_hbm.at[kid[k]], gbuf[k], gsem[k])
        pltpu.async_copy(idx_src(k + 1), raw[k + 1], isem[k + 1])

    for j in range(NBUF):
        unit(j, j, head=(j + AHEAD < NBUF), tail=False)

    assert (n_chunks - 2 * NBUF) % NBUF == 0

    @pl.loop(NBUF, n_chunks - NBUF, step=NBUF)
    def _steady(j4):
        for b in range(NBUF):
            unit(j4 + b, b, head=False, tail=False)

    for j in range(n_chunks - NBUF, n_chunks):
        unit(j, j % NBUF, head=False, tail=(j + AHEAD >= n_chunks))

    for j in range(n_chunks - NBUF, n_chunks):
        b = j % NBUF
        pltpu.make_async_copy(sbuf[b], out_dst(j), wsem[b]).wait()


def kernel(x, lut):
    bsz, seq = x.shape
    vocab = lut.shape[0]
    x_t = x.T
    lut_p = lut.reshape(vocab // 2, 2 * D_MODEL)

    mesh = plsc.VectorSubcoreMesh(
        core_axis_name="c", subcore_axis_name="s",
        num_cores=NC, num_subcores=NS)

    run = pl.kernel(
        _emb_body,
        out_type=jax.ShapeDtypeStruct((seq * D_MODEL, bsz), jnp.float32),
        mesh=mesh,
        scratch_types=(
            [pltpu.VMEM((CHUNK,), jnp.int32)] * (3 * NBUF)
            + [pltpu.VMEM((CHUNK, 2 * D_MODEL), jnp.float32)] * NBUF
            + [pltpu.VMEM((D_MODEL, CHUNK), jnp.float32)] * NBUF
            + [pltpu.SemaphoreType.DMA] * (3 * NBUF)
        ),
        compiler_params=pltpu.CompilerParams(use_tc_tiling_on_sc=True, needs_layout_passes=False),
    )
    out2 = run(x_t, lut_p)
    return out2.reshape(seq, D_MODEL, bsz).transpose(2, 0, 1)
